# baseline (speedup 1.0000x reference)
import functools
from math import exp as np_exp

import jax
import jax.numpy as jnp
from jax import lax
from jax.experimental import pallas as pl
from jax.experimental.pallas import tpu as pltpu
from jax.experimental.pallas import tpu_sc as plsc

N = 10000
K = 32
CIN = 128
COUT = 128
NB_BASIS = 10
MAXR = 2.5
SIGMA = MAXR / NB_BASIS
INV2S2 = 1.0 / (2.0 * SIGMA * SIGMA)
E = N * K

_CH = 80
_L = 16


def _sc_gather_body(ft_hbm, xs_hbm, ys_hbm, zs_hbm, idx_hbm,
                    outf_hbm, outd_hbm,
                    idx_v, f_v, d2_v, xs_v, ys_v, zs_v, semf, nc):
    wid = lax.axis_index("s") * nc + lax.axis_index("c")
    ew = E // (nc * 16)
    nch = ew // _CH
    wbase = wid * ew

    pltpu.sync_copy(xs_hbm, xs_v)
    pltpu.sync_copy(ys_hbm, ys_v)
    pltpu.sync_copy(zs_hbm, zs_v)

    def body(c, carry):
        base = pl.multiple_of(wbase + c * _CH, 8)
        pltpu.sync_copy(idx_hbm.at[pl.ds(base, _CH)], idx_v)
        cf = pltpu.async_copy(ft_hbm.at[idx_v], f_v, semf)
        for g in range(_CH // _L):
            nbr = idx_v[pl.ds(g * _L, _L)]
            own = lax.shift_right_logical(
                base + g * _L + jnp.arange(_L, dtype=jnp.int32), 5)
            dx = plsc.load_gather(xs_v, [nbr]) - plsc.load_gather(xs_v, [own])
            dy = plsc.load_gather(ys_v, [nbr]) - plsc.load_gather(ys_v, [own])
            dz = plsc.load_gather(zs_v, [nbr]) - plsc.load_gather(zs_v, [own])
            d2_v[pl.ds(g * _L, _L)] = dx * dx + dy * dy + dz * dz
        cf.wait()
        pltpu.sync_copy(f_v, outf_hbm.at[pl.ds(base, _CH)])
        pltpu.sync_copy(d2_v, outd_hbm.at[pl.ds(base, _CH)])
        return carry

    lax.fori_loop(0, nch, body, 0)


def _sc_gather(ft, xs, ys, zs, idx):
    info = plsc.get_sparse_core_info()
    nc = info.num_cores
    mesh = plsc.VectorSubcoreMesh(core_axis_name="c", subcore_axis_name="s")
    fn = functools.partial(
        pl.kernel,
        mesh=mesh,
        out_type=(
            jax.ShapeDtypeStruct((E, CIN), jnp.float32),
            jax.ShapeDtypeStruct((E,), jnp.float32),
        ),
        scratch_types=[
            pltpu.VMEM((_CH,), jnp.int32),
            pltpu.VMEM((_CH, CIN), jnp.float32),
            pltpu.VMEM((_CH,), jnp.float32),
            pltpu.VMEM((N,), jnp.float32),
            pltpu.VMEM((N,), jnp.float32),
            pltpu.VMEM((N,), jnp.float32),
            pltpu.SemaphoreType.DMA,
        ],
        compiler_params=pltpu.CompilerParams(needs_layout_passes=False),
    )(functools.partial(_sc_gather_body, nc=nc))
    return fn(ft, xs, ys, zs, idx)


_NBLK = 40
_DELTA = MAXR / (NB_BASIS - 1)
_C5 = 5.0 * _DELTA


def _tc_body(gf_ref, d2_ref, rm_ref, w_ref, out_ref):
    d2 = d2_ref[...]
    d = jnp.minimum(jnp.sqrt(d2 + 1e-12), 6.0)
    rm = rm_ref[...]
    a_lo = jnp.exp(d * d * (-INV2S2)) * rm
    dh = d - _C5
    a_hi = jnp.exp(dh * dh * (-INV2S2)) * rm
    u = jnp.exp(d * (2.0 * INV2S2 * _DELTA))

    accs = [jnp.zeros((_NBLK, CIN), jnp.float32) for _ in range(NB_BASIS)]
    for k in range(K):
        g = gf_ref[:, k, :]
        bu = u[:, k:k + 1]
        e0 = g * a_lo[:, k:k + 1]
        e5 = g * a_hi[:, k:k + 1]
        accs[0] = accs[0] + e0
        accs[5] = accs[5] + e5
        for m in range(1, 5):
            e0 = e0 * bu
            e5 = e5 * bu
            accs[m] = accs[m] + e0
            accs[5 + m] = accs[5 + m] + e5

    acc = jnp.zeros((_NBLK, COUT), jnp.float32)
    for b in range(NB_BASIS):
        m = b % 5
        if b < 5:
            s = float(np_exp(-INV2S2 * (m * _DELTA) ** 2))
        else:
            s = float(np_exp(-INV2S2 * _DELTA * _DELTA * (10 * m + m * m)))
        acc = acc + lax.dot_general(
            accs[b] * s, w_ref[b],
            dimension_numbers=(((1,), (1,)), ((), ())),
            preferred_element_type=jnp.float32)
    out_ref[...] = acc


def _tc_compute(gf3, d2, rel_mask, w):
    grid = (N // _NBLK,)
    return pl.pallas_call(
        _tc_body,
        grid=grid,
        in_specs=[
            pl.BlockSpec((_NBLK, K, CIN), lambda i: (i, 0, 0)),
            pl.BlockSpec((_NBLK, K), lambda i: (i, 0)),
            pl.BlockSpec((_NBLK, K), lambda i: (i, 0)),
            pl.BlockSpec((NB_BASIS, COUT, CIN), lambda i: (0, 0, 0)),
        ],
        out_specs=pl.BlockSpec((_NBLK, COUT), lambda i: (i, 0)),
        out_shape=jax.ShapeDtypeStruct((N, COUT), jnp.float32),
        compiler_params=pltpu.CompilerParams(
            dimension_semantics=("arbitrary",)),
    )(gf3, d2, rel_mask, w)


def kernel(features, geometry, neighbors, rel_mask, W):
    ft = features.T
    xs = geometry[:, 0]
    ys = geometry[:, 1]
    zs = geometry[:, 2]
    idx = neighbors.reshape(-1).astype(jnp.int32)
    gf, d2 = _sc_gather(ft, xs, ys, zs, idx)
    outT = _tc_compute(gf.reshape(N, K, CIN), d2.reshape(N, K), rel_mask, W)
    return outT.T


# speedup vs baseline: 3.9488x; 3.9488x over previous
import functools
from math import exp as np_exp

import jax
import jax.numpy as jnp
from jax import lax
from jax.experimental import pallas as pl
from jax.experimental.pallas import tpu as pltpu
from jax.experimental.pallas import tpu_sc as plsc

N = 10000
K = 32
CIN = 128
COUT = 128
NB_BASIS = 10
MAXR = 2.5
SIGMA = MAXR / NB_BASIS
INV2S2 = 1.0 / (2.0 * SIGMA * SIGMA)
E = N * K

_CH = 80
_L = 16


def _sc_gather_body(ft_hbm, xs_hbm, ys_hbm, zs_hbm, idx_hbm,
                    outf_hbm, outd_hbm,
                    idx_v, f_v, d2_v, xs_v, ys_v, zs_v, semf, nc):
    wid = lax.axis_index("s") * nc + lax.axis_index("c")
    ew = E // (nc * 16)
    nch = ew // _CH
    wbase = wid * ew

    pltpu.sync_copy(xs_hbm, xs_v)
    pltpu.sync_copy(ys_hbm, ys_v)
    pltpu.sync_copy(zs_hbm, zs_v)

    def body(c, carry):
        base = pl.multiple_of(wbase + c * _CH, 8)
        pltpu.sync_copy(idx_hbm.at[pl.ds(base, _CH)], idx_v)
        cf = pltpu.async_copy(ft_hbm.at[idx_v], f_v, semf)
        for g in range(_CH // _L):
            nbr = idx_v[pl.ds(g * _L, _L)]
            own = lax.shift_right_logical(
                base + g * _L + jnp.arange(_L, dtype=jnp.int32), 5)
            dx = plsc.load_gather(xs_v, [nbr]) - plsc.load_gather(xs_v, [own])
            dy = plsc.load_gather(ys_v, [nbr]) - plsc.load_gather(ys_v, [own])
            dz = plsc.load_gather(zs_v, [nbr]) - plsc.load_gather(zs_v, [own])
            d2_v[pl.ds(g * _L, _L)] = dx * dx + dy * dy + dz * dz
        cf.wait()
        pltpu.sync_copy(f_v, outf_hbm.at[pl.ds(base, _CH)])
        pltpu.sync_copy(d2_v, outd_hbm.at[pl.ds(base, _CH)])
        return carry

    lax.fori_loop(0, nch, body, 0)


def _sc_gather(ft, xs, ys, zs, idx):
    info = plsc.get_sparse_core_info()
    nc = info.num_cores
    mesh = plsc.VectorSubcoreMesh(core_axis_name="c", subcore_axis_name="s")
    fn = functools.partial(
        pl.kernel,
        mesh=mesh,
        out_type=(
            jax.ShapeDtypeStruct((E, CIN), jnp.float32),
            jax.ShapeDtypeStruct((E,), jnp.float32),
        ),
        scratch_types=[
            pltpu.VMEM((_CH,), jnp.int32),
            pltpu.VMEM((_CH, CIN), jnp.float32),
            pltpu.VMEM((_CH,), jnp.float32),
            pltpu.VMEM((N,), jnp.float32),
            pltpu.VMEM((N,), jnp.float32),
            pltpu.VMEM((N,), jnp.float32),
            pltpu.SemaphoreType.DMA,
        ],
        compiler_params=pltpu.CompilerParams(needs_layout_passes=False),
    )(functools.partial(_sc_gather_body, nc=nc))
    return fn(ft, xs, ys, zs, idx)


_NBLK = 200
_GN = 8
_CON = _GN * K
_G = _NBLK // _GN
_ROWS = _GN * NB_BASIS
_DELTA = MAXR / (NB_BASIS - 1)
_C5 = 5.0 * _DELTA


def _tc_body(gf_ref, d2_ref, rm_ref, wbig_ref, out_ref):
    d2 = d2_ref[0]
    d = jnp.minimum(jnp.sqrt(d2 + 1e-12), 6.0)
    rm = rm_ref[0]
    a_lo = jnp.exp(d * d * (-INV2S2)) * rm
    dh = d - _C5
    a_hi = jnp.exp(dh * dh * (-INV2S2)) * rm
    u = jnp.exp(d * (2.0 * INV2S2 * _DELTA))

    rbs = []
    e = a_lo
    rbs.append(e)
    for m in range(1, 5):
        e = e * u
        rbs.append(e * float(np_exp(-INV2S2 * (m * _DELTA) ** 2)))
    e = a_hi
    rbs.append(e)
    for m in range(1, 5):
        e = e * u
        rbs.append(e * float(np_exp(-INV2S2 * _DELTA * _DELTA
                                    * (10 * m + m * m))))

    rbstack = jnp.stack(rbs, axis=1)
    tiled = jnp.broadcast_to(
        rbstack[:, None, :, :], (_G, _GN, NB_BASIS, _CON)
    ).reshape(_G, _ROWS, _CON)
    ri = lax.broadcasted_iota(jnp.int32, (_ROWS, _CON), 0)
    ci = lax.broadcasted_iota(jnp.int32, (_ROWS, _CON), 1)
    maskf = (ri // NB_BASIS == ci // K).astype(jnp.float32)
    lhs = (tiled * maskf[None]).astype(jnp.bfloat16)

    rhs = gf_ref[...].astype(jnp.bfloat16)
    s = lax.dot_general(
        lhs, rhs,
        dimension_numbers=(((2,), (1,)), ((0,), (0,))),
        preferred_element_type=jnp.float32)
    s_flat = s.reshape(_NBLK, NB_BASIS * CIN).astype(jnp.bfloat16)
    out_ref[...] = lax.dot_general(
        s_flat, wbig_ref[...],
        dimension_numbers=(((1,), (0,)), ((), ())),
        preferred_element_type=jnp.float32)


def _tc_compute(gf3, d2g, rmg, wbig):
    grid = (N // _NBLK,)
    return pl.pallas_call(
        _tc_body,
        grid=grid,
        in_specs=[
            pl.BlockSpec((_G, _CON, CIN), lambda i: (i, 0, 0)),
            pl.BlockSpec((1, _G, _CON), lambda i: (i, 0, 0)),
            pl.BlockSpec((1, _G, _CON), lambda i: (i, 0, 0)),
            pl.BlockSpec((NB_BASIS * CIN, COUT), lambda i: (0, 0)),
        ],
        out_specs=pl.BlockSpec((_NBLK, COUT), lambda i: (i, 0)),
        out_shape=jax.ShapeDtypeStruct((N, COUT), jnp.float32),
        compiler_params=pltpu.CompilerParams(
            dimension_semantics=("arbitrary",)),
    )(gf3, d2g, rmg, wbig)


def kernel(features, geometry, neighbors, rel_mask, W):
    ft = features.T
    xs = geometry[:, 0]
    ys = geometry[:, 1]
    zs = geometry[:, 2]
    idx = neighbors.reshape(-1).astype(jnp.int32)
    gf, d2 = _sc_gather(ft, xs, ys, zs, idx)
    wbig = jnp.transpose(W, (0, 2, 1)).reshape(
        NB_BASIS * CIN, COUT).astype(jnp.bfloat16)
    outT = _tc_compute(gf.reshape(E // _CON, _CON, CIN),
                       d2.reshape(N // _NBLK, _G, _CON),
                       rel_mask.reshape(N // _NBLK, _G, _CON), wbig)
    return outT.T


# speedup vs baseline: 6.5208x; 1.6513x over previous
import functools
from math import exp as np_exp

import jax
import jax.numpy as jnp
from jax import lax
from jax.experimental import pallas as pl
from jax.experimental.pallas import tpu as pltpu
from jax.experimental.pallas import tpu_sc as plsc

N = 10000
K = 32
CIN = 128
COUT = 128
NB_BASIS = 10
MAXR = 2.5
SIGMA = MAXR / NB_BASIS
INV2S2 = 1.0 / (2.0 * SIGMA * SIGMA)
E = N * K

_CH = 80
_L = 16


_RING = 4
_EW = E // 32
_NCH = _EW // _CH
_NPASS = (_NCH + _RING - 1) // _RING


def _sc_gather_body(ft_hbm, xs_hbm, ys_hbm, zs_hbm, idx_hbm,
                    outf_hbm, outd_hbm,
                    idx_all, f0, f1, f2, f3, d0, d1, d2b, d3,
                    xs_v, ys_v, zs_v,
                    sf0, sf1, sf2, sf3, ss0, ss1, ss2, ss3, nc):
    wid = lax.axis_index("s") * nc + lax.axis_index("c")
    wbase = wid * _EW
    fb = [f0, f1, f2, f3]
    db = [d0, d1, d2b, d3]
    sf = [sf0, sf1, sf2, sf3]
    ss = [ss0, ss1, ss2, ss3]

    pltpu.sync_copy(idx_hbm.at[pl.ds(pl.multiple_of(wbase, 8), _EW)], idx_all)
    pltpu.sync_copy(xs_hbm, xs_v)
    pltpu.sync_copy(ys_hbm, ys_v)
    pltpu.sync_copy(zs_hbm, zs_v)

    def idx_slice(c):
        return idx_all.at[pl.ds(c * _CH, _CH)]

    def ebase(c):
        return pl.multiple_of(wbase + c * _CH, 8)

    def stores_wait(b):
        base0 = pl.multiple_of(wbase, 8)
        pltpu.make_async_copy(fb[b], outf_hbm.at[pl.ds(base0, _CH)],
                              ss[b]).wait()
        pltpu.make_async_copy(db[b], outd_hbm.at[pl.ds(base0, _CH)],
                              ss[b]).wait()

    def body(p, carry):
        for b in range(_RING):
            c = _RING * p + b

            @pl.when(c < _NCH)
            def _():
                @pl.when(c >= _RING)
                def _():
                    stores_wait(b)
                pltpu.async_copy(ft_hbm.at[idx_slice(c)], fb[b], sf[b])

        for b in range(_RING):
            c = _RING * p + b

            @pl.when(c < _NCH)
            def _():
                for g in range(_CH // _L):
                    off = c * _CH + g * _L
                    nbr = idx_all[pl.ds(off, _L)]
                    own = lax.shift_right_logical(
                        wbase + off + jnp.arange(_L, dtype=jnp.int32), 5)
                    dx = (plsc.load_gather(xs_v, [nbr])
                          - plsc.load_gather(xs_v, [own]))
                    dy = (plsc.load_gather(ys_v, [nbr])
                          - plsc.load_gather(ys_v, [own]))
                    dz = (plsc.load_gather(zs_v, [nbr])
                          - plsc.load_gather(zs_v, [own]))
                    db[b][pl.ds(g * _L, _L)] = dx * dx + dy * dy + dz * dz
                pltpu.make_async_copy(ft_hbm.at[idx_slice(c)], fb[b],
                                      sf[b]).wait()
                base = ebase(c)
                pltpu.async_copy(fb[b], outf_hbm.at[pl.ds(base, _CH)], ss[b])
                pltpu.async_copy(db[b], outd_hbm.at[pl.ds(base, _CH)], ss[b])
        return carry

    lax.fori_loop(0, _NPASS, body, 0)
    for b in range(_RING):
        stores_wait(b)


def _sc_gather(ft, xs, ys, zs, idx):
    info = plsc.get_sparse_core_info()
    nc = info.num_cores
    mesh = plsc.VectorSubcoreMesh(core_axis_name="c", subcore_axis_name="s")
    fn = functools.partial(
        pl.kernel,
        mesh=mesh,
        out_type=(
            jax.ShapeDtypeStruct((E, CIN), jnp.float32),
            jax.ShapeDtypeStruct((E,), jnp.float32),
        ),
        scratch_types=(
            [pltpu.VMEM((_EW,), jnp.int32)]
            + [pltpu.VMEM((_CH, CIN), jnp.float32)] * _RING
            + [pltpu.VMEM((_CH,), jnp.float32)] * _RING
            + [pltpu.VMEM((N,), jnp.float32)] * 3
            + [pltpu.SemaphoreType.DMA] * (2 * _RING)
        ),
        compiler_params=pltpu.CompilerParams(needs_layout_passes=False),
    )(functools.partial(_sc_gather_body, nc=nc))
    return fn(ft, xs, ys, zs, idx)


_NBLK = 200
_GN = 8
_CON = _GN * K
_G = _NBLK // _GN
_ROWS = _GN * NB_BASIS
_DELTA = MAXR / (NB_BASIS - 1)
_C5 = 5.0 * _DELTA


def _tc_body(gf_ref, d2_ref, rm_ref, wbig_ref, out_ref):
    d2 = d2_ref[0]
    d = jnp.minimum(jnp.sqrt(d2 + 1e-12), 6.0)
    rm = rm_ref[0]
    a_lo = jnp.exp(d * d * (-INV2S2)) * rm
    dh = d - _C5
    a_hi = jnp.exp(dh * dh * (-INV2S2)) * rm
    u = jnp.exp(d * (2.0 * INV2S2 * _DELTA))

    rbs = []
    e = a_lo
    rbs.append(e)
    for m in range(1, 5):
        e = e * u
        rbs.append(e * float(np_exp(-INV2S2 * (m * _DELTA) ** 2)))
    e = a_hi
    rbs.append(e)
    for m in range(1, 5):
        e = e * u
        rbs.append(e * float(np_exp(-INV2S2 * _DELTA * _DELTA
                                    * (10 * m + m * m))))

    rbstack = jnp.stack(rbs, axis=1)
    tiled = jnp.broadcast_to(
        rbstack[:, None, :, :], (_G, _GN, NB_BASIS, _CON)
    ).reshape(_G, _ROWS, _CON)
    ri = lax.broadcasted_iota(jnp.int32, (_ROWS, _CON), 0)
    ci = lax.broadcasted_iota(jnp.int32, (_ROWS, _CON), 1)
    maskf = (ri // NB_BASIS == ci // K).astype(jnp.float32)
    lhs = (tiled * maskf[None]).astype(jnp.bfloat16)

    rhs = gf_ref[...].astype(jnp.bfloat16)
    s = lax.dot_general(
        lhs, rhs,
        dimension_numbers=(((2,), (1,)), ((0,), (0,))),
        preferred_element_type=jnp.float32)
    s_flat = s.reshape(_NBLK, NB_BASIS * CIN).astype(jnp.bfloat16)
    out_ref[...] = lax.dot_general(
        s_flat, wbig_ref[...],
        dimension_numbers=(((1,), (0,)), ((), ())),
        preferred_element_type=jnp.float32)


def _tc_compute(gf3, d2g, rmg, wbig):
    grid = (N // _NBLK,)
    return pl.pallas_call(
        _tc_body,
        grid=grid,
        in_specs=[
            pl.BlockSpec((_G, _CON, CIN), lambda i: (i, 0, 0)),
            pl.BlockSpec((1, _G, _CON), lambda i: (i, 0, 0)),
            pl.BlockSpec((1, _G, _CON), lambda i: (i, 0, 0)),
            pl.BlockSpec((NB_BASIS * CIN, COUT), lambda i: (0, 0)),
        ],
        out_specs=pl.BlockSpec((_NBLK, COUT), lambda i: (i, 0)),
        out_shape=jax.ShapeDtypeStruct((N, COUT), jnp.float32),
        compiler_params=pltpu.CompilerParams(
            dimension_semantics=("arbitrary",)),
    )(gf3, d2g, rmg, wbig)


def kernel(features, geometry, neighbors, rel_mask, W):
    ft = features.T
    xs = geometry[:, 0]
    ys = geometry[:, 1]
    zs = geometry[:, 2]
    idx = neighbors.reshape(-1).astype(jnp.int32)
    gf, d2 = _sc_gather(ft, xs, ys, zs, idx)
    wbig = jnp.transpose(W, (0, 2, 1)).reshape(
        NB_BASIS * CIN, COUT).astype(jnp.bfloat16)
    outT = _tc_compute(gf.reshape(E // _CON, _CON, CIN),
                       d2.reshape(N // _NBLK, _G, _CON),
                       rel_mask.reshape(N // _NBLK, _G, _CON), wbig)
    return outT.T
